# baseline (speedup 1.0000x reference)
import jax
import jax.numpy as jnp
from jax import lax
from jax.experimental import pallas as pl
from jax.experimental.pallas import tpu as pltpu
from jax.experimental.pallas import tpu_sc as plsc


def _sc_body(x_hbm, out_hbm, v, ov):
    cid = lax.axis_index("c")
    sid = lax.axis_index("s")

    @pl.when(jnp.logical_and(cid == 0, sid == 0))
    def _():
        pltpu.sync_copy(x_hbm, v)

        lane = lax.iota(jnp.int32, 16)
        j_of = lane % 4

        kd = [plsc.load_gather(v, [4 + 4 * j_of + d]) for d in range(4)]
        c4 = (kd[0] + kd[1] + kd[2] + kd[3]) * 10.0

        head_idx = jnp.where(lane < 4, lane, 20)
        head = plsc.load_gather(v, [head_idx])
        loss = 20.0 * jnp.sum(head)

        mask_lo = lane < 4
        numers = jnp.zeros((16,), jnp.float32)
        denoms = jnp.ones((16,), jnp.float32)
        for i in range(3):
            s_i = jnp.zeros((16,), jnp.float32)
            for d in range(4):
                qd = plsc.load_gather(
                    v, [jnp.full((16,), 8 + 4 * i + d, jnp.int32)]
                )
                s_i = s_i + qd * kd[d]
            s_i = jnp.where(mask_lo, s_i * 5.0, -1e30)
            m_i = jnp.max(s_i)
            e_i = jnp.exp(s_i - m_i)
            numers = jnp.where(lane == i, jnp.sum(e_i * c4), numers)
            denoms = jnp.where(lane == i, jnp.sum(e_i), denoms)
        loss = loss + jnp.sum(jnp.where(lane < 3, numers / denoms, 0.0))

        ov[...] = jnp.full((16,), loss, jnp.float32)
        pltpu.sync_copy(ov, out_hbm)


_sc_call = pl.kernel(
    _sc_body,
    out_type=jax.ShapeDtypeStruct((16,), jnp.float32),
    mesh=plsc.VectorSubcoreMesh(core_axis_name="c", subcore_axis_name="s"),
    scratch_types=[
        pltpu.VMEM((32,), jnp.float32),
        pltpu.VMEM((16,), jnp.float32),
    ],
    compiler_params=pltpu.CompilerParams(needs_layout_passes=False),
)


@jax.jit
def kernel(base_tensor):
    flat = jnp.concatenate(
        [jnp.reshape(base_tensor, (20,)), jnp.zeros((12,), jnp.float32)]
    )
    return _sc_call(flat)[0]


# speedup vs baseline: 1.0498x; 1.0498x over previous
import jax
import jax.numpy as jnp
from jax import lax
from jax.experimental import pallas as pl
from jax.experimental.pallas import tpu as pltpu
from jax.experimental.pallas import tpu_sc as plsc


def _sc_body(x_hbm, out_hbm, v, ov):
    pltpu.sync_copy(x_hbm, v)

    lane = lax.iota(jnp.int32, 16)
    j_of = lane % 4
    zero = jnp.zeros((16,), jnp.int32)

    kd = [plsc.load_gather(v, [1 + j_of, zero + d]) for d in range(4)]
    c4 = (kd[0] + kd[1] + kd[2] + kd[3]) * 10.0

    mask_lo = lane < 4
    head = plsc.load_gather(v, [zero, jnp.where(mask_lo, lane, 0)])
    loss = 20.0 * jnp.sum(jnp.where(mask_lo, head, 0.0))

    numers = jnp.zeros((16,), jnp.float32)
    denoms = jnp.ones((16,), jnp.float32)
    for i in range(3):
        s_i = jnp.zeros((16,), jnp.float32)
        for d in range(4):
            qd = plsc.load_gather(v, [zero + (2 + i), zero + d])
            s_i = s_i + qd * kd[d]
        s_i = jnp.where(mask_lo, s_i * 5.0, -1e30)
        m_i = jnp.max(s_i)
        e_i = jnp.exp(s_i - m_i)
        numers = jnp.where(lane == i, jnp.sum(e_i * c4), numers)
        denoms = jnp.where(lane == i, jnp.sum(e_i), denoms)
    loss = loss + jnp.sum(jnp.where(lane < 3, numers / denoms, 0.0))

    ov[...] = jnp.full((16,), loss, jnp.float32)
    pltpu.sync_copy(ov.at[pl.ds(0, 1)], out_hbm)


_sc_call = pl.kernel(
    _sc_body,
    out_type=jax.ShapeDtypeStruct((1,), jnp.float32),
    mesh=plsc.VectorSubcoreMesh(
        core_axis_name="c", subcore_axis_name="s", num_cores=1, num_subcores=1
    ),
    scratch_types=[
        pltpu.VMEM((5, 4), jnp.float32),
        pltpu.VMEM((16,), jnp.float32),
    ],
    compiler_params=pltpu.CompilerParams(needs_layout_passes=False),
)


@jax.jit
def kernel(base_tensor):
    return jnp.reshape(_sc_call(base_tensor), ())
